# Initial kernel scaffold; baseline (speedup 1.0000x reference)
import jax
import jax.numpy as jnp
from jax.experimental import pallas as pl


def kernel(query_pattern, patterns):
    raise NotImplementedError("write your pallas kernel here")


# speedup vs baseline: 1.4393x; 1.4393x over previous
import functools

import jax
import jax.numpy as jnp
from jax import lax
from jax.experimental import pallas as pl
from jax.experimental.pallas import tpu as pltpu

_N = 16384
_D = 2048
_TEMP = 0.1
_THRESH = 0.85
_K = 100
_EPS = 1e-8

_BLK = 512
_STEPS = _N // _BLK
_ROWS128 = _N // 128


def _body(qcol_ref, pblk_ref, pany_ref, outp_ref, outv_ref, sims_sc, sem):
    i = pl.program_id(0)

    qcol = qcol_ref[...]
    blk = pblk_ref[...]
    dotq = lax.dot_general(blk, qcol, (((1,), (0,)), ((), ())),
                           preferred_element_type=jnp.float32)
    sq = blk * blk
    ones = jnp.ones((_D, 1), dtype=jnp.float32)
    ssq = lax.dot_general(sq, ones, (((1,), (0,)), ((), ())),
                          preferred_element_type=jnp.float32)
    qn = jnp.maximum(jnp.sqrt(jnp.sum(qcol * qcol)), _EPS)
    pn = jnp.maximum(jnp.sqrt(ssq), _EPS)
    s = (dotq / (pn * qn)) / _TEMP
    sims_sc[pl.ds(i * (_BLK // 128), _BLK // 128), :] = s.reshape(_BLK // 128, 128)

    @pl.when(i == _STEPS - 1)
    def _extract():
        outp_ref[...] = jnp.zeros_like(outp_ref)
        outv_ref[...] = jnp.zeros_like(outv_ref)
        lin = (lax.broadcasted_iota(jnp.int32, (128, 128), 0) * 128
               + lax.broadcasted_iota(jnp.int32, (128, 128), 1))
        lid = lax.broadcasted_iota(jnp.int32, (1, 128), 1)
        big = jnp.int32(2 ** 30)

        m0 = jnp.max(sims_sc[...])

        def cond(c):
            k, m = c
            return (k < _K) & (m >= _THRESH)

        def body(c):
            k, m = c
            s = sims_sc[...]
            idx = jnp.min(jnp.where(s == m, lin, big))
            outv_ref[...] = jnp.where(lid == k, m, outv_ref[...])
            sims_sc[...] = jnp.where(lin == idx, -jnp.inf, s)
            cp = pltpu.make_async_copy(
                pany_ref.at[pl.ds(idx, 1)], outp_ref.at[pl.ds(k, 1)], sem)
            cp.start()
            cp.wait()
            return k + 1, jnp.max(sims_sc[...])

        lax.while_loop(cond, body, (jnp.int32(0), m0))


@functools.partial(jax.jit, static_argnames=("interpret",))
def _run(query_pattern, patterns, interpret=False):
    qcol = query_pattern.reshape(_D, 1)
    outp, outv = pl.pallas_call(
        _body,
        grid=(_STEPS,),
        in_specs=[
            pl.BlockSpec((_D, 1), lambda i: (0, 0)),
            pl.BlockSpec((_BLK, _D), lambda i: (i, 0)),
            pl.BlockSpec(memory_space=pl.ANY),
        ],
        out_specs=[
            pl.BlockSpec((128, _D), lambda i: (0, 0)),
            pl.BlockSpec((1, 128), lambda i: (0, 0)),
        ],
        out_shape=[
            jax.ShapeDtypeStruct((128, _D), jnp.float32),
            jax.ShapeDtypeStruct((1, 128), jnp.float32),
        ],
        scratch_shapes=[
            pltpu.VMEM((128, 128), jnp.float32),
            pltpu.SemaphoreType.DMA,
        ],
        interpret=interpret,
    )(qcol, patterns, patterns)
    return outp[:_K], outv[0, :_K]


def kernel(query_pattern, patterns):
    return _run(query_pattern, patterns)
